# Initial kernel scaffold; baseline (speedup 1.0000x reference)
import jax
import jax.numpy as jnp
from jax.experimental import pallas as pl


def kernel(graph, item_feat, user_emb, W1, b1, W2, b2):
    raise NotImplementedError("write your pallas kernel here")


# speedup vs baseline: 4.3730x; 4.3730x over previous
import functools

import jax
import jax.numpy as jnp
from jax import lax
from jax.experimental import pallas as pl
from jax.experimental.pallas import tpu as pltpu
from jax.experimental.pallas import tpu_sc as plsc

N_USERS = 5000
N_ITEMS = 5000
N = N_USERS + N_ITEMS
E = 320000
FEAT = 256
HID = 128

NC = 2
NS = 16
NW = NC * NS

C = 80
E_PER_TILE = E // NW
LAYER_CHUNKS = E_PER_TILE // C
E_PER_TILE_DEG = E // NS
DEG_CHUNKS = E_PER_TILE_DEG // C
ROWS_BIG = 640
ROWS_LAST = N - 15 * ROWS_BIG


def _deg_body(graph, deg_out, zbuf, onesb, idx, deg_sh):
    c = lax.axis_index("c")
    s = lax.axis_index("s")
    zero16 = jnp.zeros((16,), jnp.float32)
    one16 = jnp.ones((16,), jnp.float32)

    @pl.loop(0, ROWS_BIG // 16)
    def _zb(i):
        zbuf[pl.ds(i * 16, 16)] = zero16

    @pl.loop(0, C // 16)
    def _ob(i):
        onesb[pl.ds(i * 16, 16)] = one16

    @pl.when(s < 15)
    def _zero_big():
        pltpu.sync_copy(zbuf, deg_sh.at[pl.ds(s * ROWS_BIG, ROWS_BIG)])

    @pl.when(s == 15)
    def _zero_last():
        pltpu.sync_copy(zbuf.at[pl.ds(0, ROWS_LAST)],
                        deg_sh.at[pl.ds(15 * ROWS_BIG, ROWS_LAST)])

    plsc.subcore_barrier()

    @pl.loop(0, DEG_CHUNKS)
    def _chunk(j):
        base = c * E + s * E_PER_TILE_DEG + j * C
        pltpu.sync_copy(graph.at[pl.ds(base, C)], idx)
        pltpu.sync_copy(onesb, deg_sh.at[idx], add=True)

    plsc.subcore_barrier()

    @pl.when(s < 15)
    def _copy_big():
        pltpu.sync_copy(deg_sh.at[pl.ds(s * ROWS_BIG, ROWS_BIG)], zbuf)
        pltpu.sync_copy(zbuf,
                        deg_out.at[pl.ds(c * N + s * ROWS_BIG, ROWS_BIG)])

    @pl.when(s == 15)
    def _copy_last():
        pltpu.sync_copy(deg_sh.at[pl.ds(15 * ROWS_BIG, ROWS_LAST)],
                        zbuf.at[pl.ds(0, ROWS_LAST)])
        pltpu.sync_copy(zbuf.at[pl.ds(0, ROWS_LAST)],
                        deg_out.at[pl.ds(c * N + 15 * ROWS_BIG, ROWS_LAST)])


@functools.lru_cache(maxsize=None)
def _get_deg_kernel():
    mesh = plsc.VectorSubcoreMesh(core_axis_name="c", subcore_axis_name="s",
                                  num_cores=NC, num_subcores=NS)
    return pl.kernel(
        _deg_body,
        out_type=jax.ShapeDtypeStruct((2 * N,), jnp.float32),
        mesh=mesh,
        scratch_types=[
            pltpu.VMEM((ROWS_BIG,), jnp.float32),
            pltpu.VMEM((C,), jnp.float32),
            pltpu.VMEM((C,), jnp.int32),
            pltpu.VMEM_SHARED((N,), jnp.float32),
        ],
    )


def _layer_body(graph, node_f, part_out, rows, idx_s, idx_d, accum, sem):
    c = lax.axis_index("c")
    s = lax.axis_index("s")
    zero16 = jnp.zeros((16,), jnp.float32)

    @pl.loop(0, C)
    def _zero_rows(i):
        for k in range(HID // 16):
            rows[i, pl.ds(k * 16, 16)] = zero16

    nz = jnp.where(s < 15, ROWS_BIG // C, ROWS_LAST // C)

    @pl.loop(0, nz)
    def _zero_acc(t):
        pltpu.sync_copy(rows, accum.at[pl.ds(s * ROWS_BIG + t * C, C)])

    plsc.subcore_barrier()

    @pl.loop(0, LAYER_CHUNKS)
    def _chunk(j):
        base = (c * NS + s) * E_PER_TILE + j * C
        pltpu.sync_copy(graph.at[pl.ds(base, C)], idx_s)
        pltpu.sync_copy(graph.at[pl.ds(E + base, C)], idx_d)
        pltpu.async_copy(node_f.at[idx_s], rows, sem).wait()
        pltpu.sync_copy(rows, accum.at[idx_d], add=True)

    plsc.subcore_barrier()

    @pl.when(s < 15)
    def _copy_big():
        pltpu.sync_copy(accum.at[pl.ds(s * ROWS_BIG, ROWS_BIG)],
                        part_out.at[c, pl.ds(s * ROWS_BIG, ROWS_BIG)])

    @pl.when(s == 15)
    def _copy_last():
        pltpu.sync_copy(accum.at[pl.ds(15 * ROWS_BIG, ROWS_LAST)],
                        part_out.at[c, pl.ds(15 * ROWS_BIG, ROWS_LAST)])


@functools.lru_cache(maxsize=None)
def _get_layer_kernel():
    mesh = plsc.VectorSubcoreMesh(core_axis_name="c", subcore_axis_name="s",
                                  num_cores=NC, num_subcores=NS)
    return pl.kernel(
        _layer_body,
        out_type=jax.ShapeDtypeStruct((2, N, HID), jnp.float32),
        mesh=mesh,
        scratch_types=[
            pltpu.VMEM((C, HID), jnp.float32),
            pltpu.VMEM((C,), jnp.int32),
            pltpu.VMEM((C,), jnp.int32),
            pltpu.VMEM_SHARED((N, HID), jnp.float32),
            pltpu.SemaphoreType.DMA,
        ],
    )


def _tc_prep_body(item_feat, W1, b1, W2, b2, user_emb, dout, node_f, res_item):
    h = jnp.maximum(
        jnp.dot(item_feat[...], W1[...], preferred_element_type=jnp.float32)
        + b1[...][None, :], 0.0)
    ri = (jnp.dot(h, W2[...], preferred_element_type=jnp.float32)
          + b2[...][None, :])
    res_item[...] = ri
    nout = lax.rsqrt(jnp.maximum(dout[...], 1.0))
    nf = jnp.concatenate([user_emb[...], ri], axis=0)
    node_f[...] = nf * nout


def _tc_mid_body(part, dout, din, user_emb, res_item,
                 node_f2, res_user1, res_item1):
    emb = (part[0] + part[1]) * lax.rsqrt(jnp.maximum(din[...], 1.0))
    nout = lax.rsqrt(jnp.maximum(dout[...], 1.0))
    node_f2[...] = emb * nout
    res_user1[...] = user_emb[...] + emb[:N_USERS] * 0.5
    res_item1[...] = res_item[...] + emb[N_USERS:] * 0.5


def _tc_final_body(part, din, res_user1, res_item1, res_user, res_item):
    emb = (part[0] + part[1]) * lax.rsqrt(jnp.maximum(din[...], 1.0))
    res_user[...] = res_user1[...] + emb[:N_USERS] * (1.0 / 3.0)
    res_item[...] = res_item1[...] + emb[N_USERS:] * (1.0 / 3.0)


_tc_prep = pl.pallas_call(
    _tc_prep_body,
    out_shape=(
        jax.ShapeDtypeStruct((N, HID), jnp.float32),
        jax.ShapeDtypeStruct((N_ITEMS, HID), jnp.float32),
    ),
)

_tc_mid = pl.pallas_call(
    _tc_mid_body,
    out_shape=(
        jax.ShapeDtypeStruct((N, HID), jnp.float32),
        jax.ShapeDtypeStruct((N_USERS, HID), jnp.float32),
        jax.ShapeDtypeStruct((N_ITEMS, HID), jnp.float32),
    ),
)

_tc_final = pl.pallas_call(
    _tc_final_body,
    out_shape=(
        jax.ShapeDtypeStruct((N_USERS, HID), jnp.float32),
        jax.ShapeDtypeStruct((N_ITEMS, HID), jnp.float32),
    ),
)


def kernel(graph, item_feat, user_emb, W1, b1, W2, b2):
    graph = graph.astype(jnp.int32).reshape(2 * E)
    deg_kernel = _get_deg_kernel()
    layer_kernel = _get_layer_kernel()
    deg_flat = deg_kernel(graph)
    dout = deg_flat[:N].reshape(N, 1)
    din = deg_flat[N:].reshape(N, 1)
    node_f1, res_item0 = _tc_prep(item_feat, W1, b1, W2, b2, user_emb, dout)
    part1 = layer_kernel(graph, node_f1)
    node_f2, res_user1, res_item1 = _tc_mid(part1, dout, din,
                                            user_emb, res_item0)
    part2 = layer_kernel(graph, node_f2)
    res_user, res_item = _tc_final(part2, din, res_user1, res_item1)
    return (res_user, res_item)


# speedup vs baseline: 6.9423x; 1.5875x over previous
import functools

import jax
import jax.numpy as jnp
from jax import lax
from jax.experimental import pallas as pl
from jax.experimental.pallas import tpu as pltpu
from jax.experimental.pallas import tpu_sc as plsc

N_USERS = 5000
N_ITEMS = 5000
N = N_USERS + N_ITEMS
E = 320000
FEAT = 256
HID = 128

NC = 2
NS = 16
NW = NC * NS

C = 40
E_PER_TILE = E // NW
PH_E = E_PER_TILE // 2
PH_CHUNKS = PH_E // C
DC = 80
E_PER_TILE_DEG = E // NS
DEG_CHUNKS = E_PER_TILE_DEG // DC
ROWS_BIG = 640
ROWS_LAST = N - 15 * ROWS_BIG


def _deg_body(graph, deg_out, zbuf, onesb, idx, deg_sh):
    c = lax.axis_index("c")
    s = lax.axis_index("s")
    zero16 = jnp.zeros((16,), jnp.float32)
    one16 = jnp.ones((16,), jnp.float32)

    @pl.loop(0, ROWS_BIG // 16)
    def _zb(i):
        zbuf[pl.ds(i * 16, 16)] = zero16

    @pl.loop(0, DC // 16)
    def _ob(i):
        onesb[pl.ds(i * 16, 16)] = one16

    @pl.when(s < 15)
    def _zero_big():
        pltpu.sync_copy(zbuf, deg_sh.at[pl.ds(s * ROWS_BIG, ROWS_BIG)])

    @pl.when(s == 15)
    def _zero_last():
        pltpu.sync_copy(zbuf.at[pl.ds(0, ROWS_LAST)],
                        deg_sh.at[pl.ds(15 * ROWS_BIG, ROWS_LAST)])

    plsc.subcore_barrier()

    @pl.loop(0, DEG_CHUNKS)
    def _chunk(j):
        base = c * E + s * E_PER_TILE_DEG + j * DC
        pltpu.sync_copy(graph.at[pl.ds(base, DC)], idx)
        pltpu.sync_copy(onesb, deg_sh.at[idx], add=True)

    plsc.subcore_barrier()

    @pl.when(s < 15)
    def _copy_big():
        pltpu.sync_copy(deg_sh.at[pl.ds(s * ROWS_BIG, ROWS_BIG)], zbuf)
        pltpu.sync_copy(zbuf,
                        deg_out.at[pl.ds(c * N + s * ROWS_BIG, ROWS_BIG)])

    @pl.when(s == 15)
    def _copy_last():
        pltpu.sync_copy(deg_sh.at[pl.ds(15 * ROWS_BIG, ROWS_LAST)],
                        zbuf.at[pl.ds(0, ROWS_LAST)])
        pltpu.sync_copy(zbuf.at[pl.ds(0, ROWS_LAST)],
                        deg_out.at[pl.ds(c * N + 15 * ROWS_BIG, ROWS_LAST)])


@functools.lru_cache(maxsize=None)
def _get_deg_kernel():
    mesh = plsc.VectorSubcoreMesh(core_axis_name="c", subcore_axis_name="s",
                                  num_cores=NC, num_subcores=NS)
    return pl.kernel(
        _deg_body,
        out_type=jax.ShapeDtypeStruct((2 * N,), jnp.float32),
        mesh=mesh,
        scratch_types=[
            pltpu.VMEM((ROWS_BIG,), jnp.float32),
            pltpu.VMEM((DC,), jnp.float32),
            pltpu.VMEM((DC,), jnp.int32),
            pltpu.VMEM_SHARED((N,), jnp.float32),
        ],
    )


def _layer_body(graph, node_f, part_out, rows0, rows1, src1, dst2d, accum,
                isem, gsem0, gsem1):
    c = lax.axis_index("c")
    s = lax.axis_index("s")
    zero16 = jnp.zeros((16,), jnp.float32)
    base0 = (c * NS + s) * E_PER_TILE

    @pl.loop(0, C)
    def _zero_rows(i):
        for k in range(HID // 16):
            rows0[i, pl.ds(k * 16, 16)] = zero16

    nz = jnp.where(s < 15, ROWS_BIG // C, ROWS_LAST // C)

    @pl.loop(0, nz)
    def _zero_acc(t):
        pltpu.sync_copy(rows0, accum.at[pl.ds(s * ROWS_BIG + t * C, C)])

    plsc.subcore_barrier()

    def _gather(j, rows, gsem):
        pltpu.async_copy(node_f.at[src1.at[pl.ds(j * C, C)]], rows, gsem)

    def _wait_gather(rows, gsem):
        pltpu.make_async_copy(node_f.at[src1.at[pl.ds(0, C)]], rows,
                              gsem).wait()

    def _scatter(j, rows):
        pltpu.sync_copy(rows, accum.at[dst2d.at[j]], add=True)

    for ph in range(2):
        ebase = base0 + ph * PH_E

        pltpu.async_copy(graph.at[pl.ds(ebase, PH_E)], src1, isem)

        @pl.loop(0, PH_CHUNKS)
        def _fire_idx(j):
            pltpu.async_copy(graph.at[pl.ds(E + ebase + j * C, C)],
                             dst2d.at[j], isem)

        pltpu.make_async_copy(graph.at[pl.ds(ebase, PH_E)], src1,
                              isem).wait()

        @pl.loop(0, PH_CHUNKS)
        def _drain_idx(j):
            pltpu.make_async_copy(graph.at[pl.ds(E + ebase + j * C, C)],
                                  dst2d.at[j], isem).wait()

        _gather(0, rows0, gsem0)
        _gather(1, rows1, gsem1)

        cnt = PH_CHUNKS
        npair = (cnt - 1) // 2 if cnt % 2 else (cnt - 2) // 2

        @pl.loop(0, npair)
        def _pair(t):
            j = 2 * t
            _wait_gather(rows0, gsem0)
            _scatter(j, rows0)
            _gather(j + 2, rows0, gsem0)

            _wait_gather(rows1, gsem1)
            _scatter(j + 1, rows1)

            @pl.when(j + 3 < cnt)
            def _prefetch_odd():
                _gather(j + 3, rows1, gsem1)

        if cnt % 2:
            _wait_gather(rows0, gsem0)
            _scatter(cnt - 1, rows0)
        else:
            _wait_gather(rows0, gsem0)
            _scatter(cnt - 2, rows0)
            _wait_gather(rows1, gsem1)
            _scatter(cnt - 1, rows1)

    plsc.subcore_barrier()

    @pl.when(s < 15)
    def _copy_big():
        pltpu.sync_copy(accum.at[pl.ds(s * ROWS_BIG, ROWS_BIG)],
                        part_out.at[c, pl.ds(s * ROWS_BIG, ROWS_BIG)])

    @pl.when(s == 15)
    def _copy_last():
        pltpu.sync_copy(accum.at[pl.ds(15 * ROWS_BIG, ROWS_LAST)],
                        part_out.at[c, pl.ds(15 * ROWS_BIG, ROWS_LAST)])


@functools.lru_cache(maxsize=None)
def _get_layer_kernel():
    mesh = plsc.VectorSubcoreMesh(core_axis_name="c", subcore_axis_name="s",
                                  num_cores=NC, num_subcores=NS)
    return pl.kernel(
        _layer_body,
        out_type=jax.ShapeDtypeStruct((2, N, HID), jnp.float32),
        mesh=mesh,
        scratch_types=[
            pltpu.VMEM((C, HID), jnp.float32),
            pltpu.VMEM((C, HID), jnp.float32),
            pltpu.VMEM((PH_E,), jnp.int32),
            pltpu.VMEM((PH_CHUNKS, C), jnp.int32),
            pltpu.VMEM_SHARED((N, HID), jnp.float32),
            pltpu.SemaphoreType.DMA,
            pltpu.SemaphoreType.DMA,
            pltpu.SemaphoreType.DMA,
        ],
    )


def _tc_prep_body(item_feat, W1, b1, W2, b2, user_emb, dout, node_f, res_item):
    h = jnp.maximum(
        jnp.dot(item_feat[...], W1[...], preferred_element_type=jnp.float32)
        + b1[...][None, :], 0.0)
    ri = (jnp.dot(h, W2[...], preferred_element_type=jnp.float32)
          + b2[...][None, :])
    res_item[...] = ri
    nout = lax.rsqrt(jnp.maximum(dout[...], 1.0))
    nf = jnp.concatenate([user_emb[...], ri], axis=0)
    node_f[...] = nf * nout


def _tc_mid_body(part, dout, din, user_emb, res_item,
                 node_f2, res_user1, res_item1):
    emb = (part[0] + part[1]) * lax.rsqrt(jnp.maximum(din[...], 1.0))
    nout = lax.rsqrt(jnp.maximum(dout[...], 1.0))
    node_f2[...] = emb * nout
    res_user1[...] = user_emb[...] + emb[:N_USERS] * 0.5
    res_item1[...] = res_item[...] + emb[N_USERS:] * 0.5


def _tc_final_body(part, din, res_user1, res_item1, res_user, res_item):
    emb = (part[0] + part[1]) * lax.rsqrt(jnp.maximum(din[...], 1.0))
    res_user[...] = res_user1[...] + emb[:N_USERS] * (1.0 / 3.0)
    res_item[...] = res_item1[...] + emb[N_USERS:] * (1.0 / 3.0)


_tc_prep = pl.pallas_call(
    _tc_prep_body,
    out_shape=(
        jax.ShapeDtypeStruct((N, HID), jnp.float32),
        jax.ShapeDtypeStruct((N_ITEMS, HID), jnp.float32),
    ),
)

_tc_mid = pl.pallas_call(
    _tc_mid_body,
    out_shape=(
        jax.ShapeDtypeStruct((N, HID), jnp.float32),
        jax.ShapeDtypeStruct((N_USERS, HID), jnp.float32),
        jax.ShapeDtypeStruct((N_ITEMS, HID), jnp.float32),
    ),
)

_tc_final = pl.pallas_call(
    _tc_final_body,
    out_shape=(
        jax.ShapeDtypeStruct((N_USERS, HID), jnp.float32),
        jax.ShapeDtypeStruct((N_ITEMS, HID), jnp.float32),
    ),
)


def kernel(graph, item_feat, user_emb, W1, b1, W2, b2):
    graph = graph.astype(jnp.int32).reshape(2 * E)
    deg_kernel = _get_deg_kernel()
    layer_kernel = _get_layer_kernel()
    deg_flat = deg_kernel(graph)
    dout = deg_flat[:N].reshape(N, 1)
    din = deg_flat[N:].reshape(N, 1)
    node_f1, res_item0 = _tc_prep(item_feat, W1, b1, W2, b2, user_emb, dout)
    part1 = layer_kernel(graph, node_f1)
    node_f2, res_user1, res_item1 = _tc_mid(part1, dout, din,
                                            user_emb, res_item0)
    part2 = layer_kernel(graph, node_f2)
    res_user, res_item = _tc_final(part2, din, res_user1, res_item1)
    return (res_user, res_item)


# speedup vs baseline: 9.0811x; 1.3081x over previous
import functools

import jax
import jax.numpy as jnp
from jax import lax
from jax.experimental import pallas as pl
from jax.experimental.pallas import tpu as pltpu
from jax.experimental.pallas import tpu_sc as plsc

N_USERS = 5000
N_ITEMS = 5000
N = N_USERS + N_ITEMS
E = 320000
FEAT = 256
HID = 128

NC = 2
NS = 16
NW = NC * NS

C = 40
E_PER_TILE = E // NW
PH_E = E_PER_TILE // 2
PH_CHUNKS = PH_E // C
DC = 80
E_PER_TILE_DEG = E // NS
DEG_CHUNKS = E_PER_TILE_DEG // DC
ROWS_BIG = 640
ROWS_LAST = N - 15 * ROWS_BIG


def _deg_body(graph, deg_out, zbuf, onesb, idx2d, deg_sh, isem, ssem):
    c = lax.axis_index("c")
    s = lax.axis_index("s")
    zero16 = jnp.zeros((16,), jnp.float32)
    one16 = jnp.ones((16,), jnp.float32)
    base0 = c * E + s * E_PER_TILE_DEG

    @pl.loop(0, DEG_CHUNKS)
    def _fire_idx(j):
        pltpu.async_copy(graph.at[pl.ds(base0 + j * DC, DC)], idx2d.at[j],
                         isem)

    @pl.loop(0, ROWS_BIG // 16)
    def _zb(i):
        zbuf[pl.ds(i * 16, 16)] = zero16

    @pl.loop(0, DC // 16)
    def _ob(i):
        onesb[pl.ds(i * 16, 16)] = one16

    @pl.when(s < 15)
    def _zero_big():
        pltpu.sync_copy(zbuf, deg_sh.at[pl.ds(s * ROWS_BIG, ROWS_BIG)])

    @pl.when(s == 15)
    def _zero_last():
        pltpu.sync_copy(zbuf.at[pl.ds(0, ROWS_LAST)],
                        deg_sh.at[pl.ds(15 * ROWS_BIG, ROWS_LAST)])

    @pl.loop(0, DEG_CHUNKS)
    def _drain_idx(j):
        pltpu.make_async_copy(graph.at[pl.ds(base0 + j * DC, DC)],
                              idx2d.at[j], isem).wait()

    plsc.subcore_barrier()

    @pl.loop(0, DEG_CHUNKS)
    def _chunk(j):
        pltpu.async_copy(onesb, deg_sh.at[idx2d.at[j]], ssem, add=True)

    @pl.loop(0, DEG_CHUNKS)
    def _drain_sc(j):
        pltpu.make_async_copy(onesb, deg_sh.at[idx2d.at[j]], ssem).wait()

    plsc.subcore_barrier()

    @pl.when(s < 15)
    def _copy_big():
        pltpu.sync_copy(deg_sh.at[pl.ds(s * ROWS_BIG, ROWS_BIG)], zbuf)
        pltpu.sync_copy(zbuf,
                        deg_out.at[pl.ds(c * N + s * ROWS_BIG, ROWS_BIG)])

    @pl.when(s == 15)
    def _copy_last():
        pltpu.sync_copy(deg_sh.at[pl.ds(15 * ROWS_BIG, ROWS_LAST)],
                        zbuf.at[pl.ds(0, ROWS_LAST)])
        pltpu.sync_copy(zbuf.at[pl.ds(0, ROWS_LAST)],
                        deg_out.at[pl.ds(c * N + 15 * ROWS_BIG, ROWS_LAST)])


@functools.lru_cache(maxsize=None)
def _get_deg_kernel():
    mesh = plsc.VectorSubcoreMesh(core_axis_name="c", subcore_axis_name="s",
                                  num_cores=NC, num_subcores=NS)
    return pl.kernel(
        _deg_body,
        out_type=jax.ShapeDtypeStruct((2 * N,), jnp.float32),
        mesh=mesh,
        scratch_types=[
            pltpu.VMEM((ROWS_BIG,), jnp.float32),
            pltpu.VMEM((DC,), jnp.float32),
            pltpu.VMEM((DEG_CHUNKS, DC), jnp.int32),
            pltpu.VMEM_SHARED((N,), jnp.float32),
            pltpu.SemaphoreType.DMA,
            pltpu.SemaphoreType.DMA,
        ],
    )


def _layer_body(graph, node_f, part_out, rows0, rows1, src1, dst2d, accum,
                isem, gsem0, gsem1):
    c = lax.axis_index("c")
    s = lax.axis_index("s")
    zero16 = jnp.zeros((16,), jnp.float32)
    base0 = (c * NS + s) * E_PER_TILE

    @pl.loop(0, C)
    def _zero_rows(i):
        for k in range(HID // 16):
            rows0[i, pl.ds(k * 16, 16)] = zero16

    nz = jnp.where(s < 15, ROWS_BIG // C, ROWS_LAST // C)

    @pl.loop(0, nz)
    def _zero_acc(t):
        pltpu.sync_copy(rows0, accum.at[pl.ds(s * ROWS_BIG + t * C, C)])

    plsc.subcore_barrier()

    def _gather(j, rows, gsem):
        pltpu.async_copy(node_f.at[src1.at[pl.ds(j * C, C)]], rows, gsem)

    def _wait_gather(rows, gsem):
        pltpu.make_async_copy(node_f.at[src1.at[pl.ds(0, C)]], rows,
                              gsem).wait()

    def _scatter(j, rows):
        pltpu.sync_copy(rows, accum.at[dst2d.at[j]], add=True)

    for ph in range(2):
        ebase = base0 + ph * PH_E

        pltpu.async_copy(graph.at[pl.ds(ebase, PH_E)], src1, isem)

        @pl.loop(0, PH_CHUNKS)
        def _fire_idx(j):
            pltpu.async_copy(graph.at[pl.ds(E + ebase + j * C, C)],
                             dst2d.at[j], isem)

        pltpu.make_async_copy(graph.at[pl.ds(ebase, PH_E)], src1,
                              isem).wait()

        @pl.loop(0, PH_CHUNKS)
        def _drain_idx(j):
            pltpu.make_async_copy(graph.at[pl.ds(E + ebase + j * C, C)],
                                  dst2d.at[j], isem).wait()

        _gather(0, rows0, gsem0)
        _gather(1, rows1, gsem1)

        cnt = PH_CHUNKS
        npair = (cnt - 1) // 2 if cnt % 2 else (cnt - 2) // 2

        @pl.loop(0, npair)
        def _pair(t):
            j = 2 * t
            _wait_gather(rows0, gsem0)
            _scatter(j, rows0)
            _gather(j + 2, rows0, gsem0)

            _wait_gather(rows1, gsem1)
            _scatter(j + 1, rows1)

            @pl.when(j + 3 < cnt)
            def _prefetch_odd():
                _gather(j + 3, rows1, gsem1)

        if cnt % 2:
            _wait_gather(rows0, gsem0)
            _scatter(cnt - 1, rows0)
        else:
            _wait_gather(rows0, gsem0)
            _scatter(cnt - 2, rows0)
            _wait_gather(rows1, gsem1)
            _scatter(cnt - 1, rows1)

    plsc.subcore_barrier()

    @pl.when(s < 15)
    def _copy_big():
        pltpu.sync_copy(accum.at[pl.ds(s * ROWS_BIG, ROWS_BIG)],
                        part_out.at[c, pl.ds(s * ROWS_BIG, ROWS_BIG)])

    @pl.when(s == 15)
    def _copy_last():
        pltpu.sync_copy(accum.at[pl.ds(15 * ROWS_BIG, ROWS_LAST)],
                        part_out.at[c, pl.ds(15 * ROWS_BIG, ROWS_LAST)])


@functools.lru_cache(maxsize=None)
def _get_layer_kernel():
    mesh = plsc.VectorSubcoreMesh(core_axis_name="c", subcore_axis_name="s",
                                  num_cores=NC, num_subcores=NS)
    return pl.kernel(
        _layer_body,
        out_type=jax.ShapeDtypeStruct((2, N, HID), jnp.float32),
        mesh=mesh,
        scratch_types=[
            pltpu.VMEM((C, HID), jnp.float32),
            pltpu.VMEM((C, HID), jnp.float32),
            pltpu.VMEM((PH_E,), jnp.int32),
            pltpu.VMEM((PH_CHUNKS, C), jnp.int32),
            pltpu.VMEM_SHARED((N, HID), jnp.float32),
            pltpu.SemaphoreType.DMA,
            pltpu.SemaphoreType.DMA,
            pltpu.SemaphoreType.DMA,
        ],
    )


def _tc_prep_body(item_feat, W1, b1, W2, b2, user_emb, dout, node_f, res_item):
    h = jnp.maximum(
        jnp.dot(item_feat[...], W1[...], preferred_element_type=jnp.float32)
        + b1[...][None, :], 0.0)
    ri = (jnp.dot(h, W2[...], preferred_element_type=jnp.float32)
          + b2[...][None, :])
    res_item[...] = ri
    nout = lax.rsqrt(jnp.maximum(dout[...], 1.0))
    nf = jnp.concatenate([user_emb[...], ri], axis=0)
    node_f[...] = nf * nout


def _tc_mid_body(part, dout, din, user_emb, res_item,
                 node_f2, res_user1, res_item1):
    emb = (part[0] + part[1]) * lax.rsqrt(jnp.maximum(din[...], 1.0))
    nout = lax.rsqrt(jnp.maximum(dout[...], 1.0))
    node_f2[...] = emb * nout
    res_user1[...] = user_emb[...] + emb[:N_USERS] * 0.5
    res_item1[...] = res_item[...] + emb[N_USERS:] * 0.5


def _tc_final_body(part, din, res_user1, res_item1, res_user, res_item):
    emb = (part[0] + part[1]) * lax.rsqrt(jnp.maximum(din[...], 1.0))
    res_user[...] = res_user1[...] + emb[:N_USERS] * (1.0 / 3.0)
    res_item[...] = res_item1[...] + emb[N_USERS:] * (1.0 / 3.0)


_tc_prep = pl.pallas_call(
    _tc_prep_body,
    out_shape=(
        jax.ShapeDtypeStruct((N, HID), jnp.float32),
        jax.ShapeDtypeStruct((N_ITEMS, HID), jnp.float32),
    ),
)

_tc_mid = pl.pallas_call(
    _tc_mid_body,
    out_shape=(
        jax.ShapeDtypeStruct((N, HID), jnp.float32),
        jax.ShapeDtypeStruct((N_USERS, HID), jnp.float32),
        jax.ShapeDtypeStruct((N_ITEMS, HID), jnp.float32),
    ),
)

_tc_final = pl.pallas_call(
    _tc_final_body,
    out_shape=(
        jax.ShapeDtypeStruct((N_USERS, HID), jnp.float32),
        jax.ShapeDtypeStruct((N_ITEMS, HID), jnp.float32),
    ),
)


def kernel(graph, item_feat, user_emb, W1, b1, W2, b2):
    graph = graph.astype(jnp.int32).reshape(2 * E)
    deg_kernel = _get_deg_kernel()
    layer_kernel = _get_layer_kernel()
    deg_flat = deg_kernel(graph)
    dout = deg_flat[:N].reshape(N, 1)
    din = deg_flat[N:].reshape(N, 1)
    node_f1, res_item0 = _tc_prep(item_feat, W1, b1, W2, b2, user_emb, dout)
    part1 = layer_kernel(graph, node_f1)
    node_f2, res_user1, res_item1 = _tc_mid(part1, dout, din,
                                            user_emb, res_item0)
    part2 = layer_kernel(graph, node_f2)
    res_user, res_item = _tc_final(part2, din, res_user1, res_item1)
    return (res_user, res_item)


# speedup vs baseline: 11.6731x; 1.2854x over previous
import functools

import jax
import jax.numpy as jnp
from jax import lax
from jax.experimental import pallas as pl
from jax.experimental.pallas import tpu as pltpu
from jax.experimental.pallas import tpu_sc as plsc

N_USERS = 5000
N_ITEMS = 5000
N = N_USERS + N_ITEMS
E = 320000
FEAT = 256
HID = 128

NC = 2
NS = 16
NW = NC * NS

C = 40
E_PER_TILE = E // NW
PH_E = E_PER_TILE // 2
PH_CHUNKS = PH_E // C
DC = 80
E_PER_TILE_DEG = E // NS
DEG_CHUNKS = E_PER_TILE_DEG // DC
ROWS_BIG = 640
ROWS_LAST = N - 15 * ROWS_BIG


def _deg_body(graph, deg_out, zbuf, onesb, idx2d, deg_sh, isem, ssem):
    c = lax.axis_index("c")
    s = lax.axis_index("s")
    zero16 = jnp.zeros((16,), jnp.float32)
    one16 = jnp.ones((16,), jnp.float32)
    base0 = c * E + s * E_PER_TILE_DEG

    @pl.loop(0, DEG_CHUNKS)
    def _fire_idx(j):
        pltpu.async_copy(graph.at[pl.ds(base0 + j * DC, DC)], idx2d.at[j],
                         isem)

    @pl.loop(0, ROWS_BIG // 16)
    def _zb(i):
        zbuf[pl.ds(i * 16, 16)] = zero16

    @pl.loop(0, DC // 16)
    def _ob(i):
        onesb[pl.ds(i * 16, 16)] = one16

    @pl.when(s < 15)
    def _zero_big():
        pltpu.sync_copy(zbuf, deg_sh.at[pl.ds(s * ROWS_BIG, ROWS_BIG)])

    @pl.when(s == 15)
    def _zero_last():
        pltpu.sync_copy(zbuf.at[pl.ds(0, ROWS_LAST)],
                        deg_sh.at[pl.ds(15 * ROWS_BIG, ROWS_LAST)])

    @pl.loop(0, DEG_CHUNKS)
    def _drain_idx(j):
        pltpu.make_async_copy(graph.at[pl.ds(base0 + j * DC, DC)],
                              idx2d.at[j], isem).wait()

    plsc.subcore_barrier()

    @pl.loop(0, DEG_CHUNKS)
    def _chunk(j):
        pltpu.async_copy(onesb, deg_sh.at[idx2d.at[j]], ssem, add=True)

    @pl.loop(0, DEG_CHUNKS)
    def _drain_sc(j):
        pltpu.make_async_copy(onesb, deg_sh.at[idx2d.at[j]], ssem).wait()

    plsc.subcore_barrier()

    @pl.when(s < 15)
    def _copy_big():
        pltpu.sync_copy(deg_sh.at[pl.ds(s * ROWS_BIG, ROWS_BIG)], zbuf)
        pltpu.sync_copy(zbuf,
                        deg_out.at[pl.ds(c * N + s * ROWS_BIG, ROWS_BIG)])

    @pl.when(s == 15)
    def _copy_last():
        pltpu.sync_copy(deg_sh.at[pl.ds(15 * ROWS_BIG, ROWS_LAST)],
                        zbuf.at[pl.ds(0, ROWS_LAST)])
        pltpu.sync_copy(zbuf.at[pl.ds(0, ROWS_LAST)],
                        deg_out.at[pl.ds(c * N + 15 * ROWS_BIG, ROWS_LAST)])


@functools.lru_cache(maxsize=None)
def _get_deg_kernel():
    mesh = plsc.VectorSubcoreMesh(core_axis_name="c", subcore_axis_name="s",
                                  num_cores=NC, num_subcores=NS)
    return pl.kernel(
        _deg_body,
        out_type=jax.ShapeDtypeStruct((2 * N,), jnp.float32),
        mesh=mesh,
        scratch_types=[
            pltpu.VMEM((ROWS_BIG,), jnp.float32),
            pltpu.VMEM((DC,), jnp.float32),
            pltpu.VMEM((DEG_CHUNKS, DC), jnp.int32),
            pltpu.VMEM_SHARED((N,), jnp.float32),
            pltpu.SemaphoreType.DMA,
            pltpu.SemaphoreType.DMA,
        ],
    )


def _layer_body(graph, node_f, part_out, rows0, rows1, rows2, src1, dst2d,
                accum, isem, gsem0, gsem1, gsem2, ssem0, ssem1, ssem2):
    c = lax.axis_index("c")
    s = lax.axis_index("s")
    zero16 = jnp.zeros((16,), jnp.float32)
    base0 = (c * NS + s) * E_PER_TILE

    rows = (rows0, rows1, rows2)
    gsem = (gsem0, gsem1, gsem2)
    ssem = (ssem0, ssem1, ssem2)

    @pl.loop(0, C)
    def _zero_rows(i):
        for k in range(HID // 16):
            rows0[i, pl.ds(k * 16, 16)] = zero16

    nz = jnp.where(s < 15, ROWS_BIG // C, ROWS_LAST // C)

    @pl.loop(0, nz)
    def _zero_acc(t):
        pltpu.async_copy(rows0, accum.at[pl.ds(s * ROWS_BIG + t * C, C)],
                         gsem0)

    def _fire_phase_idx(ebase):
        pltpu.async_copy(graph.at[pl.ds(ebase, PH_E)], src1, isem)

        @pl.loop(0, PH_CHUNKS)
        def _fire_idx(j):
            pltpu.async_copy(graph.at[pl.ds(E + ebase + j * C, C)],
                             dst2d.at[j], isem)

    def _drain_phase_idx(ebase):
        pltpu.make_async_copy(graph.at[pl.ds(ebase, PH_E)], src1,
                              isem).wait()

        @pl.loop(0, PH_CHUNKS)
        def _drain_idx(j):
            pltpu.make_async_copy(graph.at[pl.ds(E + ebase + j * C, C)],
                                  dst2d.at[j], isem).wait()

    _fire_phase_idx(base0)

    @pl.loop(0, nz)
    def _drain_zero(t):
        pltpu.make_async_copy(rows0, accum.at[pl.ds(s * ROWS_BIG + t * C, C)],
                              gsem0).wait()

    _drain_phase_idx(base0)

    plsc.subcore_barrier()

    def _gather(j, b):
        pltpu.async_copy(node_f.at[src1.at[pl.ds(j * C, C)]], rows[b],
                         gsem[b])

    def _wait_gather(b):
        pltpu.make_async_copy(node_f.at[src1.at[pl.ds(0, C)]], rows[b],
                              gsem[b]).wait()

    def _scatter(j, b):
        pltpu.async_copy(rows[b], accum.at[dst2d.at[j]], ssem[b], add=True)

    def _wait_scatter(b):
        pltpu.make_async_copy(rows[b], accum.at[dst2d.at[0]], ssem[b]).wait()

    cnt = PH_CHUNKS
    ntrip = (cnt - 2) // 3
    for ph in range(2):
        ebase = base0 + ph * PH_E
        if ph > 0:
            _fire_phase_idx(ebase)
            _drain_phase_idx(ebase)

        for b in range(3):
            _gather(b, b)

        @pl.loop(0, ntrip)
        def _trip(t):
            for b in range(3):
                j = 3 * t + b
                _wait_gather(b)
                _scatter(j, b)

                @pl.when(j + 3 < cnt)
                def _refill():
                    _wait_scatter(b)
                    _gather(j + 3, b)

        _wait_gather(0)
        _scatter(cnt - 2, 0)
        _wait_gather(1)
        _scatter(cnt - 1, 1)
        for b in range(3):
            _wait_scatter(b)

    plsc.subcore_barrier()

    @pl.when(s < 15)
    def _copy_big():
        pltpu.sync_copy(accum.at[pl.ds(s * ROWS_BIG, ROWS_BIG)],
                        part_out.at[c, pl.ds(s * ROWS_BIG, ROWS_BIG)])

    @pl.when(s == 15)
    def _copy_last():
        pltpu.sync_copy(accum.at[pl.ds(15 * ROWS_BIG, ROWS_LAST)],
                        part_out.at[c, pl.ds(15 * ROWS_BIG, ROWS_LAST)])


@functools.lru_cache(maxsize=None)
def _get_layer_kernel():
    mesh = plsc.VectorSubcoreMesh(core_axis_name="c", subcore_axis_name="s",
                                  num_cores=NC, num_subcores=NS)
    return pl.kernel(
        _layer_body,
        out_type=jax.ShapeDtypeStruct((2, N, HID), jnp.float32),
        mesh=mesh,
        scratch_types=[
            pltpu.VMEM((C, HID), jnp.float32),
            pltpu.VMEM((C, HID), jnp.float32),
            pltpu.VMEM((C, HID), jnp.float32),
            pltpu.VMEM((PH_E,), jnp.int32),
            pltpu.VMEM((PH_CHUNKS, C), jnp.int32),
            pltpu.VMEM_SHARED((N, HID), jnp.float32),
            pltpu.SemaphoreType.DMA,
            pltpu.SemaphoreType.DMA,
            pltpu.SemaphoreType.DMA,
            pltpu.SemaphoreType.DMA,
            pltpu.SemaphoreType.DMA,
            pltpu.SemaphoreType.DMA,
            pltpu.SemaphoreType.DMA,
        ],
    )


def _tc_prep_body(item_feat, W1, b1, W2, b2, user_emb, dout, node_f, res_item):
    h = jnp.maximum(
        jnp.dot(item_feat[...], W1[...], preferred_element_type=jnp.float32)
        + b1[...][None, :], 0.0)
    ri = (jnp.dot(h, W2[...], preferred_element_type=jnp.float32)
          + b2[...][None, :])
    res_item[...] = ri
    nout = lax.rsqrt(jnp.maximum(dout[...], 1.0))
    nf = jnp.concatenate([user_emb[...], ri], axis=0)
    node_f[...] = nf * nout


def _tc_mid_body(part, dout, din, user_emb, res_item,
                 node_f2, res_user1, res_item1):
    emb = (part[0] + part[1]) * lax.rsqrt(jnp.maximum(din[...], 1.0))
    nout = lax.rsqrt(jnp.maximum(dout[...], 1.0))
    node_f2[...] = emb * nout
    res_user1[...] = user_emb[...] + emb[:N_USERS] * 0.5
    res_item1[...] = res_item[...] + emb[N_USERS:] * 0.5


def _tc_final_body(part, din, res_user1, res_item1, res_user, res_item):
    emb = (part[0] + part[1]) * lax.rsqrt(jnp.maximum(din[...], 1.0))
    res_user[...] = res_user1[...] + emb[:N_USERS] * (1.0 / 3.0)
    res_item[...] = res_item1[...] + emb[N_USERS:] * (1.0 / 3.0)


_tc_prep = pl.pallas_call(
    _tc_prep_body,
    out_shape=(
        jax.ShapeDtypeStruct((N, HID), jnp.float32),
        jax.ShapeDtypeStruct((N_ITEMS, HID), jnp.float32),
    ),
)

_tc_mid = pl.pallas_call(
    _tc_mid_body,
    out_shape=(
        jax.ShapeDtypeStruct((N, HID), jnp.float32),
        jax.ShapeDtypeStruct((N_USERS, HID), jnp.float32),
        jax.ShapeDtypeStruct((N_ITEMS, HID), jnp.float32),
    ),
)

_tc_final = pl.pallas_call(
    _tc_final_body,
    out_shape=(
        jax.ShapeDtypeStruct((N_USERS, HID), jnp.float32),
        jax.ShapeDtypeStruct((N_ITEMS, HID), jnp.float32),
    ),
)


def kernel(graph, item_feat, user_emb, W1, b1, W2, b2):
    graph = graph.astype(jnp.int32).reshape(2 * E)
    deg_kernel = _get_deg_kernel()
    layer_kernel = _get_layer_kernel()
    deg_flat = deg_kernel(graph)
    dout = deg_flat[:N].reshape(N, 1)
    din = deg_flat[N:].reshape(N, 1)
    node_f1, res_item0 = _tc_prep(item_feat, W1, b1, W2, b2, user_emb, dout)
    part1 = layer_kernel(graph, node_f1)
    node_f2, res_user1, res_item1 = _tc_mid(part1, dout, din,
                                            user_emb, res_item0)
    part2 = layer_kernel(graph, node_f2)
    res_user, res_item = _tc_final(part2, din, res_user1, res_item1)
    return (res_user, res_item)


# speedup vs baseline: 12.6409x; 1.0829x over previous
import functools

import jax
import jax.numpy as jnp
from jax import lax
from jax.experimental import pallas as pl
from jax.experimental.pallas import tpu as pltpu
from jax.experimental.pallas import tpu_sc as plsc

N_USERS = 5000
N_ITEMS = 5000
N = N_USERS + N_ITEMS
E = 320000
FEAT = 256
HID = 128

NC = 2
NS = 16
NW = NC * NS

C = 40
E_PER_TILE = E // NW
N_PHASES = 5
PH_E = E_PER_TILE // N_PHASES
PH_CHUNKS = PH_E // C
DEPTH = 4
DC = 80
E_PER_TILE_DEG = E // NS
DEG_CHUNKS = E_PER_TILE_DEG // DC
ROWS_BIG = 640
ROWS_LAST = N - 15 * ROWS_BIG


def _deg_body(graph, deg_out, zbuf, onesb, idx2d, deg_sh, isem, ssem):
    c = lax.axis_index("c")
    s = lax.axis_index("s")
    zero16 = jnp.zeros((16,), jnp.float32)
    one16 = jnp.ones((16,), jnp.float32)
    base0 = c * E + s * E_PER_TILE_DEG

    @pl.loop(0, DEG_CHUNKS)
    def _fire_idx(j):
        pltpu.async_copy(graph.at[pl.ds(base0 + j * DC, DC)], idx2d.at[j],
                         isem)

    @pl.loop(0, ROWS_BIG // 16)
    def _zb(i):
        zbuf[pl.ds(i * 16, 16)] = zero16

    @pl.loop(0, DC // 16)
    def _ob(i):
        onesb[pl.ds(i * 16, 16)] = one16

    @pl.when(s < 15)
    def _zero_big():
        pltpu.sync_copy(zbuf, deg_sh.at[pl.ds(s * ROWS_BIG, ROWS_BIG)])

    @pl.when(s == 15)
    def _zero_last():
        pltpu.sync_copy(zbuf.at[pl.ds(0, ROWS_LAST)],
                        deg_sh.at[pl.ds(15 * ROWS_BIG, ROWS_LAST)])

    @pl.loop(0, DEG_CHUNKS)
    def _drain_idx(j):
        pltpu.make_async_copy(graph.at[pl.ds(base0 + j * DC, DC)],
                              idx2d.at[j], isem).wait()

    plsc.subcore_barrier()

    @pl.loop(0, DEG_CHUNKS)
    def _chunk(j):
        pltpu.async_copy(onesb, deg_sh.at[idx2d.at[j]], ssem, add=True)

    @pl.loop(0, DEG_CHUNKS)
    def _drain_sc(j):
        pltpu.make_async_copy(onesb, deg_sh.at[idx2d.at[j]], ssem).wait()

    plsc.subcore_barrier()

    @pl.when(s < 15)
    def _copy_big():
        pltpu.sync_copy(deg_sh.at[pl.ds(s * ROWS_BIG, ROWS_BIG)], zbuf)
        pltpu.sync_copy(zbuf,
                        deg_out.at[pl.ds(c * N + s * ROWS_BIG, ROWS_BIG)])

    @pl.when(s == 15)
    def _copy_last():
        pltpu.sync_copy(deg_sh.at[pl.ds(15 * ROWS_BIG, ROWS_LAST)],
                        zbuf.at[pl.ds(0, ROWS_LAST)])
        pltpu.sync_copy(zbuf.at[pl.ds(0, ROWS_LAST)],
                        deg_out.at[pl.ds(c * N + 15 * ROWS_BIG, ROWS_LAST)])


@functools.lru_cache(maxsize=None)
def _get_deg_kernel():
    mesh = plsc.VectorSubcoreMesh(core_axis_name="c", subcore_axis_name="s",
                                  num_cores=NC, num_subcores=NS)
    return pl.kernel(
        _deg_body,
        out_type=jax.ShapeDtypeStruct((2 * N,), jnp.float32),
        mesh=mesh,
        scratch_types=[
            pltpu.VMEM((ROWS_BIG,), jnp.float32),
            pltpu.VMEM((DC,), jnp.float32),
            pltpu.VMEM((DEG_CHUNKS, DC), jnp.int32),
            pltpu.VMEM_SHARED((N,), jnp.float32),
            pltpu.SemaphoreType.DMA,
            pltpu.SemaphoreType.DMA,
        ],
    )


def _layer_body(graph, node_f, part_out, rows0, rows1, rows2, rows3, src1,
                dst2d, accum, isem, gsem0, gsem1, gsem2, gsem3,
                ssem0, ssem1, ssem2, ssem3):
    c = lax.axis_index("c")
    s = lax.axis_index("s")
    zero16 = jnp.zeros((16,), jnp.float32)
    base0 = (c * NS + s) * E_PER_TILE

    rows = (rows0, rows1, rows2, rows3)
    gsem = (gsem0, gsem1, gsem2, gsem3)
    ssem = (ssem0, ssem1, ssem2, ssem3)

    @pl.loop(0, C)
    def _zero_rows(i):
        for k in range(HID // 16):
            rows0[i, pl.ds(k * 16, 16)] = zero16

    nz = jnp.where(s < 15, ROWS_BIG // C, ROWS_LAST // C)

    @pl.loop(0, nz)
    def _zero_acc(t):
        pltpu.async_copy(rows0, accum.at[pl.ds(s * ROWS_BIG + t * C, C)],
                         gsem0)

    def _fire_phase_idx(ebase):
        pltpu.async_copy(graph.at[pl.ds(ebase, PH_E)], src1, isem)

        @pl.loop(0, PH_CHUNKS)
        def _fire_idx(j):
            pltpu.async_copy(graph.at[pl.ds(E + ebase + j * C, C)],
                             dst2d.at[j], isem)

    def _drain_phase_idx(ebase):
        pltpu.make_async_copy(graph.at[pl.ds(ebase, PH_E)], src1,
                              isem).wait()

        @pl.loop(0, PH_CHUNKS)
        def _drain_idx(j):
            pltpu.make_async_copy(graph.at[pl.ds(E + ebase + j * C, C)],
                                  dst2d.at[j], isem).wait()

    _fire_phase_idx(base0)

    @pl.loop(0, nz)
    def _drain_zero(t):
        pltpu.make_async_copy(rows0, accum.at[pl.ds(s * ROWS_BIG + t * C, C)],
                              gsem0).wait()

    _drain_phase_idx(base0)

    plsc.subcore_barrier()

    def _gather(j, b):
        pltpu.async_copy(node_f.at[src1.at[pl.ds(j * C, C)]], rows[b],
                         gsem[b])

    def _wait_gather(b):
        pltpu.make_async_copy(node_f.at[src1.at[pl.ds(0, C)]], rows[b],
                              gsem[b]).wait()

    def _scatter(j, b):
        pltpu.async_copy(rows[b], accum.at[dst2d.at[j]], ssem[b], add=True)

    def _wait_scatter(b):
        pltpu.make_async_copy(rows[b], accum.at[dst2d.at[0]], ssem[b]).wait()

    cnt = PH_CHUNKS
    ntrip = cnt // DEPTH
    nepi = cnt - ntrip * DEPTH
    for ph in range(N_PHASES):
        ebase = base0 + ph * PH_E
        if ph > 0:
            _fire_phase_idx(ebase)
            _drain_phase_idx(ebase)

        for b in range(DEPTH):
            _gather(b, b)

        @pl.loop(0, ntrip)
        def _trip(t):
            for b in range(DEPTH):
                j = DEPTH * t + b
                _wait_gather(b)
                _scatter(j, b)

                @pl.when(j + DEPTH < cnt)
                def _refill():
                    _wait_scatter(b)
                    _gather(j + DEPTH, b)

        for b in range(nepi):
            _wait_gather(b)
            _scatter(ntrip * DEPTH + b, b)
        for b in range(DEPTH):
            _wait_scatter(b)

    plsc.subcore_barrier()

    @pl.when(s < 15)
    def _copy_big():
        pltpu.sync_copy(accum.at[pl.ds(s * ROWS_BIG, ROWS_BIG)],
                        part_out.at[c, pl.ds(s * ROWS_BIG, ROWS_BIG)])

    @pl.when(s == 15)
    def _copy_last():
        pltpu.sync_copy(accum.at[pl.ds(15 * ROWS_BIG, ROWS_LAST)],
                        part_out.at[c, pl.ds(15 * ROWS_BIG, ROWS_LAST)])


@functools.lru_cache(maxsize=None)
def _get_layer_kernel():
    mesh = plsc.VectorSubcoreMesh(core_axis_name="c", subcore_axis_name="s",
                                  num_cores=NC, num_subcores=NS)
    return pl.kernel(
        _layer_body,
        out_type=jax.ShapeDtypeStruct((2, N, HID), jnp.float32),
        mesh=mesh,
        scratch_types=(
            [pltpu.VMEM((C, HID), jnp.float32)] * DEPTH
            + [
                pltpu.VMEM((PH_E,), jnp.int32),
                pltpu.VMEM((PH_CHUNKS, C), jnp.int32),
                pltpu.VMEM_SHARED((N, HID), jnp.float32),
            ]
            + [pltpu.SemaphoreType.DMA] * (1 + 2 * DEPTH)
        ),
    )


def _tc_prep_body(item_feat, W1, b1, W2, b2, user_emb, dout, node_f, res_item):
    h = jnp.maximum(
        jnp.dot(item_feat[...], W1[...], preferred_element_type=jnp.float32)
        + b1[...][None, :], 0.0)
    ri = (jnp.dot(h, W2[...], preferred_element_type=jnp.float32)
          + b2[...][None, :])
    res_item[...] = ri
    nout = lax.rsqrt(jnp.maximum(dout[...], 1.0))
    nf = jnp.concatenate([user_emb[...], ri], axis=0)
    node_f[...] = nf * nout


def _tc_mid_body(part, dout, din, user_emb, res_item,
                 node_f2, res_user1, res_item1):
    emb = (part[0] + part[1]) * lax.rsqrt(jnp.maximum(din[...], 1.0))
    nout = lax.rsqrt(jnp.maximum(dout[...], 1.0))
    node_f2[...] = emb * nout
    res_user1[...] = user_emb[...] + emb[:N_USERS] * 0.5
    res_item1[...] = res_item[...] + emb[N_USERS:] * 0.5


def _tc_final_body(part, din, res_user1, res_item1, res_user, res_item):
    emb = (part[0] + part[1]) * lax.rsqrt(jnp.maximum(din[...], 1.0))
    res_user[...] = res_user1[...] + emb[:N_USERS] * (1.0 / 3.0)
    res_item[...] = res_item1[...] + emb[N_USERS:] * (1.0 / 3.0)


_tc_prep = pl.pallas_call(
    _tc_prep_body,
    out_shape=(
        jax.ShapeDtypeStruct((N, HID), jnp.float32),
        jax.ShapeDtypeStruct((N_ITEMS, HID), jnp.float32),
    ),
)

_tc_mid = pl.pallas_call(
    _tc_mid_body,
    out_shape=(
        jax.ShapeDtypeStruct((N, HID), jnp.float32),
        jax.ShapeDtypeStruct((N_USERS, HID), jnp.float32),
        jax.ShapeDtypeStruct((N_ITEMS, HID), jnp.float32),
    ),
)

_tc_final = pl.pallas_call(
    _tc_final_body,
    out_shape=(
        jax.ShapeDtypeStruct((N_USERS, HID), jnp.float32),
        jax.ShapeDtypeStruct((N_ITEMS, HID), jnp.float32),
    ),
)


def kernel(graph, item_feat, user_emb, W1, b1, W2, b2):
    graph = graph.astype(jnp.int32).reshape(2 * E)
    deg_kernel = _get_deg_kernel()
    layer_kernel = _get_layer_kernel()
    deg_flat = deg_kernel(graph)
    dout = deg_flat[:N].reshape(N, 1)
    din = deg_flat[N:].reshape(N, 1)
    node_f1, res_item0 = _tc_prep(item_feat, W1, b1, W2, b2, user_emb, dout)
    part1 = layer_kernel(graph, node_f1)
    node_f2, res_user1, res_item1 = _tc_mid(part1, dout, din,
                                            user_emb, res_item0)
    part2 = layer_kernel(graph, node_f2)
    res_user, res_item = _tc_final(part2, din, res_user1, res_item1)
    return (res_user, res_item)
